# baseline (speedup 1.0000x reference)
import functools
import math

import jax
import jax.numpy as jnp
from jax import lax
from jax.experimental import pallas as pl
from jax.experimental.pallas import tpu as pltpu
from jax.experimental.pallas import tpu_sc as plsc

BATCH = 1024
SEQ = 200
D = 128
B = BATCH * SEQ
NC = 2
NS = 16
NW = NC * NS
B_PER_W = B // NW
CHUNK = 192
CS = [64, 128] + [192] * 32 + [64]
OFFS = [sum(CS[:i]) for i in range(len(CS))]
NCHUNK = len(CS)
assert sum(CS) == B_PER_W and all(c <= CHUNK for c in CS)
NBUF = 5
LANES = 16
SCALE = float(math.sqrt(D))


def _make_kernel():
  mesh = plsc.VectorSubcoreMesh(core_axis_name="c", subcore_axis_name="s")

  @functools.partial(
      pl.kernel,
      mesh=mesh,
      out_type=jax.ShapeDtypeStruct((B, D), jnp.float32),
      scratch_types=[
          pltpu.VMEM((B_PER_W,), jnp.int32),
          pltpu.VMEM((NBUF, CHUNK, D), jnp.float32),
          pltpu.SemaphoreType.DMA((NBUF,)),
          pltpu.SemaphoreType.DMA((NBUF,)),
          pltpu.SemaphoreType.DMA,
      ],
  )
  def emb_kernel(idx_hbm, table_hbm, out_hbm, idx_v, rows_v, gsem, wsem,
                 isem):
    wid = lax.axis_index("s") * NC + lax.axis_index("c")
    base = wid * B_PER_W
    pltpu.sync_copy(idx_hbm.at[pl.ds(base, CS[0])], idx_v.at[pl.ds(0, CS[0])])
    rest = pltpu.async_copy(
        idx_hbm.at[pl.ds(base + CS[0], B_PER_W - CS[0])],
        idx_v.at[pl.ds(CS[0], B_PER_W - CS[0])],
        isem,
    )

    def start_gather(g):
      off, c = OFFS[g], CS[g]
      return pltpu.async_copy(
          table_hbm.at[idx_v.at[pl.ds(off, c)]],
          rows_v.at[g % NBUF, pl.ds(0, c)],
          gsem.at[g % NBUF],
      )

    def scale_rows(b, lo, hi):
      def row_body(i, carry):
        for j in range(D // LANES):
          sl = pl.ds(j * LANES, LANES)
          rows_v[b, i, sl] = rows_v[b, i, sl] * SCALE
        return carry

      lax.fori_loop(lo, hi, row_body, 0)

    gh = [None] * NCHUNK
    wh = [None] * NCHUNK
    gh[0] = start_gather(0)
    rest.wait()
    gh[1] = start_gather(1)
    gh[2] = start_gather(2)
    for g in range(NCHUNK):
      if g + 3 < NCHUNK:
        if g + 3 >= NBUF:
          wh[g + 3 - NBUF].wait()
        gh[g + 3] = start_gather(g + 3)
      gh[g].wait()
      b = g % NBUF
      scale_rows(b, 0, CS[g])
      wh[g] = pltpu.async_copy(
          rows_v.at[b, pl.ds(0, CS[g])],
          out_hbm.at[pl.ds(base + OFFS[g], CS[g])],
          wsem.at[b],
      )
    for g in range(NCHUNK - NBUF, NCHUNK):
      wh[g].wait()

  return emb_kernel


_emb = _make_kernel()


def kernel(x, table):
  idx = x.reshape(-1).astype(jnp.int32)
  out = _emb(idx, table)
  return out.reshape(BATCH, SEQ, D)


# speedup vs baseline: 1.0127x; 1.0127x over previous
import functools
import math

import jax
import jax.numpy as jnp
from jax import lax
from jax.experimental import pallas as pl
from jax.experimental.pallas import tpu as pltpu
from jax.experimental.pallas import tpu_sc as plsc

BATCH = 1024
SEQ = 200
D = 128
B = BATCH * SEQ
NC = 2
NS = 16
NW = NC * NS
B_PER_W = B // NW
CHUNK = 240
CS = [48, 96, 192] + [240] * 24 + [160, 96, 48]
OFFS = [sum(CS[:i]) for i in range(len(CS))]
NCHUNK = len(CS)
assert sum(CS) == B_PER_W and all(c <= CHUNK for c in CS)
NBUF = 4
LANES = 16
SCALE = float(math.sqrt(D))


def _make_kernel():
  mesh = plsc.VectorSubcoreMesh(core_axis_name="c", subcore_axis_name="s")

  @functools.partial(
      pl.kernel,
      mesh=mesh,
      out_type=jax.ShapeDtypeStruct((B, D), jnp.float32),
      scratch_types=[
          pltpu.VMEM((B_PER_W,), jnp.int32),
          pltpu.VMEM((NBUF, CHUNK, D), jnp.float32),
          pltpu.SemaphoreType.DMA((NBUF,)),
          pltpu.SemaphoreType.DMA((NBUF,)),
          pltpu.SemaphoreType.DMA,
      ],
  )
  def emb_kernel(idx_hbm, table_hbm, out_hbm, idx_v, rows_v, gsem, wsem,
                 isem):
    wid = lax.axis_index("s") * NC + lax.axis_index("c")
    base = wid * B_PER_W
    head_n = OFFS[5]
    pltpu.sync_copy(idx_hbm.at[pl.ds(base, head_n)], idx_v.at[pl.ds(0, head_n)])
    rest = pltpu.async_copy(
        idx_hbm.at[pl.ds(base + head_n, B_PER_W - head_n)],
        idx_v.at[pl.ds(head_n, B_PER_W - head_n)],
        isem,
    )

    def start_gather(g):
      off, c = OFFS[g], CS[g]
      return pltpu.async_copy(
          table_hbm.at[idx_v.at[pl.ds(off, c)]],
          rows_v.at[g % NBUF, pl.ds(0, c)],
          gsem.at[g % NBUF],
      )

    def scale_rows(b, lo, hi):
      def row_body(i, carry):
        for j in range(D // LANES):
          sl = pl.ds(j * LANES, LANES)
          rows_v[b, i, sl] = rows_v[b, i, sl] * SCALE
        return carry

      lax.fori_loop(lo, hi, row_body, 0)

    gh = [None] * NCHUNK
    wh = [None] * NCHUNK
    gh[0] = start_gather(0)
    gh[1] = start_gather(1)
    for g in range(NCHUNK):
      if g == 3:
        rest.wait()
      if g + 2 < NCHUNK:
        if g + 2 >= NBUF:
          wh[g + 2 - NBUF].wait()
        gh[g + 2] = start_gather(g + 2)
      gh[g].wait()
      b = g % NBUF
      scale_rows(b, 0, CS[g])
      wh[g] = pltpu.async_copy(
          rows_v.at[b, pl.ds(0, CS[g])],
          out_hbm.at[pl.ds(base + OFFS[g], CS[g])],
          wsem.at[b],
      )
    for g in range(NCHUNK - NBUF, NCHUNK):
      wh[g].wait()

  return emb_kernel


_emb = _make_kernel()


def kernel(x, table):
  idx = x.reshape(-1).astype(jnp.int32)
  out = _emb(idx, table)
  return out.reshape(BATCH, SEQ, D)
